# baseline (speedup 1.0000x reference)
import functools

import jax
import jax.numpy as jnp
from jax import lax
from jax.experimental import pallas as pl
from jax.experimental.pallas import tpu as pltpu
from jax.experimental.pallas import tpu_sc as plsc

N_NODES = 10000
N_EDGES = 320000
DIM = 128
EPS = 1e-5

NC = 2
NS = 16
NW = NC * NS

EDGES_PER_W = N_EDGES // NW
CHUNK = 80
N_CHUNKS = EDGES_PER_W // CHUNK
NBUF = 4
N_PAD = 10240
ROWS_PER_TILE = N_PAD // NS
ZB_ROWS = 32


def _sc_scatter_add(dst, e):
    mesh = plsc.VectorSubcoreMesh(core_axis_name="c", subcore_axis_name="s")

    @functools.partial(
        pl.kernel,
        out_type=jax.ShapeDtypeStruct((NC, N_PAD, DIM), jnp.float32),
        mesh=mesh,
        scratch_types=[
            pltpu.VMEM((NBUF, CHUNK), jnp.int32),
            pltpu.VMEM((NBUF, CHUNK, DIM), jnp.float32),
            pltpu.VMEM((ZB_ROWS, DIM), jnp.float32),
            pltpu.VMEM_SHARED((N_PAD, DIM), jnp.float32),
            pltpu.SemaphoreType.DMA((NBUF,)),
            pltpu.SemaphoreType.DMA((NBUF,)),
        ],
    )
    def body(dst_hbm, e_hbm, out_hbm, idx, rows, zbuf, acc, sem_g, sem_s):
        cid = lax.axis_index("c")
        sid = lax.axis_index("s")
        wid = sid * NC + cid
        base = wid * EDGES_PER_W

        def fire_gather(ci, b):
            pltpu.async_copy(dst_hbm.at[wid, ci], idx.at[b], sem_g.at[b])
            pltpu.async_copy(e_hbm.at[pl.ds(base + ci * CHUNK, CHUNK)],
                             rows.at[b], sem_g.at[b])

        def drain_gather(ci, b):
            pltpu.make_async_copy(dst_hbm.at[wid, ci], idx.at[b],
                                  sem_g.at[b]).wait()
            pltpu.make_async_copy(e_hbm.at[pl.ds(base + ci * CHUNK, CHUNK)],
                                  rows.at[b], sem_g.at[b]).wait()

        def fire_scatter(b):
            pltpu.async_copy(rows.at[b], acc.at[idx.at[b]], sem_s.at[b],
                             add=True)

        def drain_scatter(b):
            pltpu.make_async_copy(rows.at[b], acc.at[idx.at[b]],
                                  sem_s.at[b]).wait()

        for c in range(NBUF):
            fire_gather(c, c)

        zeros16 = jnp.zeros((16,), jnp.float32)

        def zstore(i, carry):
            r = i // (DIM // 16)
            c = (i % (DIM // 16)) * 16
            zbuf[r, pl.ds(c, 16)] = zeros16
            return carry

        lax.fori_loop(0, ZB_ROWS * (DIM // 16), zstore, 0)

        def zcopy(k, carry):
            pltpu.sync_copy(zbuf, acc.at[pl.ds(sid * ROWS_PER_TILE + k * ZB_ROWS, ZB_ROWS)])
            return carry

        lax.fori_loop(0, ROWS_PER_TILE // ZB_ROWS, zcopy, 0)
        plsc.subcore_barrier()

        def chunk_body(c, carry):
            b = lax.rem(c, NBUF)
            drain_gather(c, b)
            fire_scatter(b)

            @pl.when(c >= 1)
            def _():
                b1 = lax.rem(c - 1, NBUF)

                @pl.when(c + NBUF - 1 < N_CHUNKS)
                def _():
                    drain_scatter(b1)
                    fire_gather(c + NBUF - 1, b1)

            return carry

        lax.fori_loop(0, N_CHUNKS, chunk_body, 0)
        for c in range(N_CHUNKS - NBUF, N_CHUNKS):
            drain_scatter(c % NBUF)
        plsc.subcore_barrier()

        def wb(k, carry):
            r = sid * ROWS_PER_TILE + k * ZB_ROWS
            pltpu.sync_copy(acc.at[pl.ds(r, ZB_ROWS)], zbuf)
            pltpu.sync_copy(zbuf, out_hbm.at[cid, pl.ds(r, ZB_ROWS)])
            return carry

        lax.fori_loop(0, ROWS_PER_TILE // ZB_ROWS, wb, 0)

    return body(dst, e)


BR = 1000


def _xpart_block(x_ref, w0x_ref, b0_ref, a0_ref):
    a0_ref[...] = (jnp.dot(x_ref[...], w0x_ref[...],
                           preferred_element_type=jnp.float32) + b0_ref[...])


def _xpart(x, w0x, b0):
    n = x.shape[0]
    row_spec = pl.BlockSpec((BR, DIM), lambda i: (i, 0))
    full = lambda a: pl.BlockSpec(a.shape, lambda i: (0,) * a.ndim)
    return pl.pallas_call(
        _xpart_block,
        grid=(n // BR,),
        in_specs=[row_spec, full(w0x), full(b0)],
        out_specs=row_spec,
        out_shape=jax.ShapeDtypeStruct((n, DIM), jnp.float32),
        compiler_params=pltpu.CompilerParams(
            dimension_semantics=("parallel",),
        ),
    )(x, w0x, b0)


def _mlp_block(x_ref, a0_ref, p_ref, w0m_ref, w1_ref, w2_ref,
               b1_ref, b2_ref, gnw_ref, gnb_ref, out_ref):
    x = x_ref[...]
    msg = p_ref[0] + p_ref[1]
    h = a0_ref[...]
    h += jnp.dot(msg, w0m_ref[...], preferred_element_type=jnp.float32)
    h = jnp.maximum(h, 0.0)
    h = jnp.dot(h, w1_ref[...], preferred_element_type=jnp.float32)
    h = jnp.maximum(h + b1_ref[...], 0.0)
    h = jnp.dot(h, w2_ref[...], preferred_element_type=jnp.float32)
    h = h + b2_ref[...]
    mean = jnp.mean(h, axis=1, keepdims=True)
    var = jnp.mean((h - mean) ** 2, axis=1, keepdims=True)
    h = (h - mean) * lax.rsqrt(var + EPS) * gnw_ref[...] + gnb_ref[...]
    out_ref[...] = x + h


def _mlp(x, a0, partials, w0m, w1t, w2t, b1, b2, gn_w, gn_b):
    n = x.shape[0]
    grid = (n // BR,)
    row_spec = pl.BlockSpec((BR, DIM), lambda i: (i, 0))
    p_spec = pl.BlockSpec((NC, BR, DIM), lambda i: (0, i, 0))
    full = lambda a: pl.BlockSpec(a.shape, lambda i: (0,) * a.ndim)
    return pl.pallas_call(
        _mlp_block,
        grid=grid,
        in_specs=[row_spec, row_spec, p_spec,
                  full(w0m), full(w1t), full(w2t),
                  full(b1), full(b2), full(gn_w), full(gn_b)],
        out_specs=row_spec,
        out_shape=jax.ShapeDtypeStruct((n, DIM), jnp.float32),
        compiler_params=pltpu.CompilerParams(
            dimension_semantics=("parallel",),
        ),
    )(x, a0, partials, w0m, w1t, w2t, b1, b2, gn_w, gn_b)


def kernel(x, edge_index, e, W0, b0, W1, b1, W2, b2, gn_w, gn_b):
    dst = edge_index[1].reshape(NW, N_CHUNKS, CHUNK)
    partials = _sc_scatter_add(dst, e)
    w0t = W0.T
    a0 = _xpart(x, w0t[:DIM], b0[None, :])
    out = _mlp(x, a0, partials,
               w0t[DIM:], W1.T, W2.T,
               b1[None, :], b2[None, :],
               gn_w[None, :], gn_b[None, :])
    return out


# speedup vs baseline: 1.0471x; 1.0471x over previous
import functools

import jax
import jax.numpy as jnp
from jax import lax
from jax.experimental import pallas as pl
from jax.experimental.pallas import tpu as pltpu
from jax.experimental.pallas import tpu_sc as plsc

N_NODES = 10000
N_EDGES = 320000
DIM = 128
EPS = 1e-5

NC = 2
NS = 16
NW = NC * NS

EDGES_PER_W = N_EDGES // NW
CHUNK = 80
N_CHUNKS = EDGES_PER_W // CHUNK
NBUF = 3
N_PAD = 10240
ROWS_PER_TILE = N_PAD // NS
WB_ROWS = CHUNK


def _sc_scatter_add(dst, e):
    mesh = plsc.VectorSubcoreMesh(core_axis_name="c", subcore_axis_name="s")

    @functools.partial(
        pl.kernel,
        out_type=jax.ShapeDtypeStruct((NC, N_PAD, DIM), jnp.float32),
        mesh=mesh,
        scratch_types=[
            pltpu.VMEM((N_CHUNKS, CHUNK), jnp.int32),
            pltpu.VMEM((NBUF, CHUNK, DIM), jnp.float32),
            pltpu.VMEM_SHARED((N_PAD, DIM), jnp.float32),
            pltpu.SemaphoreType.DMA((NBUF,)),
            pltpu.SemaphoreType.DMA((NBUF,)),
            pltpu.SemaphoreType.DMA,
            pltpu.SemaphoreType.DMA,
        ],
    )
    def body(dst_hbm, e_hbm, out_hbm, idx_all, rows, acc,
             sem_g, sem_s, sem_z, sem_i):
        cid = lax.axis_index("c")
        sid = lax.axis_index("s")
        wid = sid * NC + cid
        base = wid * EDGES_PER_W
        rbase = sid * ROWS_PER_TILE

        def fire_gather(ci, b):
            pltpu.async_copy(e_hbm.at[pl.ds(base + ci * CHUNK, CHUNK)],
                             rows.at[b], sem_g.at[b])

        def drain_gather(ci, b):
            pltpu.make_async_copy(e_hbm.at[pl.ds(base + ci * CHUNK, CHUNK)],
                                  rows.at[b], sem_g.at[b]).wait()

        def fire_scatter(ci, b):
            pltpu.async_copy(rows.at[b], acc.at[idx_all.at[ci]], sem_s.at[b],
                             add=True)

        def drain_scatter(ci, b):
            pltpu.make_async_copy(rows.at[b], acc.at[idx_all.at[ci]],
                                  sem_s.at[b]).wait()

        pltpu.async_copy(dst_hbm.at[wid], idx_all, sem_i)
        for c in range(1, NBUF):
            fire_gather(c, c)

        zeros16 = jnp.zeros((16,), jnp.float32)

        def zstore(i, carry):
            r = i // (DIM // 16)
            c = (i % (DIM // 16)) * 16
            rows[0, r, pl.ds(c, 16)] = zeros16
            return carry

        lax.fori_loop(0, CHUNK * (DIM // 16), zstore, 0)

        def zfire(k, carry):
            pltpu.async_copy(rows.at[0],
                             acc.at[pl.ds(rbase + k * CHUNK, CHUNK)], sem_z)
            return carry

        lax.fori_loop(0, ROWS_PER_TILE // CHUNK, zfire, 0)

        def zdrain(k, carry):
            pltpu.make_async_copy(
                rows.at[0], acc.at[pl.ds(rbase + k * CHUNK, CHUNK)],
                sem_z).wait()
            return carry

        lax.fori_loop(0, ROWS_PER_TILE // CHUNK, zdrain, 0)
        fire_gather(0, 0)
        pltpu.make_async_copy(dst_hbm.at[wid], idx_all, sem_i).wait()
        plsc.subcore_barrier()

        def chunk_body(c, carry):
            b = lax.rem(c, NBUF)
            drain_gather(c, b)
            fire_scatter(c, b)

            @pl.when(c >= 1)
            def _():
                b1 = lax.rem(c - 1, NBUF)

                @pl.when(c + NBUF - 1 < N_CHUNKS)
                def _():
                    drain_scatter(c - 1, b1)
                    fire_gather(c + NBUF - 1, b1)

            return carry

        lax.fori_loop(0, N_CHUNKS, chunk_body, 0)
        for c in range(N_CHUNKS - NBUF, N_CHUNKS):
            drain_scatter(c, c % NBUF)
        plsc.subcore_barrier()

        n_wb = ROWS_PER_TILE // WB_ROWS
        for k in range(n_wb):
            b = k % 2
            r = rbase + k * WB_ROWS
            if k >= 2:
                pltpu.make_async_copy(
                    rows.at[b],
                    out_hbm.at[cid, pl.ds(rbase + (k - 2) * WB_ROWS, WB_ROWS)],
                    sem_s.at[b]).wait()
            pltpu.async_copy(acc.at[pl.ds(r, WB_ROWS)], rows.at[b],
                             sem_g.at[b])
            pltpu.make_async_copy(acc.at[pl.ds(r, WB_ROWS)], rows.at[b],
                                  sem_g.at[b]).wait()
            pltpu.async_copy(rows.at[b], out_hbm.at[cid, pl.ds(r, WB_ROWS)],
                             sem_s.at[b])
        for k in range(n_wb - 2, n_wb):
            b = k % 2
            pltpu.make_async_copy(
                rows.at[b],
                out_hbm.at[cid, pl.ds(rbase + k * WB_ROWS, WB_ROWS)],
                sem_s.at[b]).wait()

    return body(dst, e)


BR = 1000


def _mlp_block(x_ref, p_ref, w0x_ref, w0m_ref, w1_ref, w2_ref,
               b0_ref, b1_ref, b2_ref, gnw_ref, gnb_ref, out_ref):
    x = x_ref[...]
    msg = p_ref[0] + p_ref[1]
    h = jnp.dot(x, w0x_ref[...], preferred_element_type=jnp.float32)
    h += jnp.dot(msg, w0m_ref[...], preferred_element_type=jnp.float32)
    h = jnp.maximum(h + b0_ref[...], 0.0)
    h = jnp.dot(h, w1_ref[...], preferred_element_type=jnp.float32)
    h = jnp.maximum(h + b1_ref[...], 0.0)
    h = jnp.dot(h, w2_ref[...], preferred_element_type=jnp.float32)
    h = h + b2_ref[...]
    mean = jnp.mean(h, axis=1, keepdims=True)
    var = jnp.mean((h - mean) ** 2, axis=1, keepdims=True)
    h = (h - mean) * lax.rsqrt(var + EPS) * gnw_ref[...] + gnb_ref[...]
    out_ref[...] = x + h


def _mlp(x, partials, w0x, w0m, w1t, w2t, b0, b1, b2, gn_w, gn_b):
    n = x.shape[0]
    grid = (n // BR,)
    row_spec = pl.BlockSpec((BR, DIM), lambda i: (i, 0))
    p_spec = pl.BlockSpec((NC, BR, DIM), lambda i: (0, i, 0))
    full = lambda a: pl.BlockSpec(a.shape, lambda i: (0,) * a.ndim)
    return pl.pallas_call(
        _mlp_block,
        grid=grid,
        in_specs=[row_spec, p_spec,
                  full(w0x), full(w0m), full(w1t), full(w2t),
                  full(b0), full(b1), full(b2), full(gn_w), full(gn_b)],
        out_specs=row_spec,
        out_shape=jax.ShapeDtypeStruct((n, DIM), jnp.float32),
        compiler_params=pltpu.CompilerParams(
            dimension_semantics=("parallel",),
        ),
    )(x, partials, w0x, w0m, w1t, w2t, b0, b1, b2, gn_w, gn_b)


def kernel(x, edge_index, e, W0, b0, W1, b1, W2, b2, gn_w, gn_b):
    dst = edge_index[1].reshape(NW, N_CHUNKS, CHUNK)
    partials = _sc_scatter_add(dst, e)
    w0t = W0.T
    out = _mlp(x, partials,
               w0t[:DIM], w0t[DIM:], W1.T, W2.T,
               b0[None, :], b1[None, :], b2[None, :],
               gn_w[None, :], gn_b[None, :])
    return out


# speedup vs baseline: 1.0732x; 1.0250x over previous
import functools

import jax
import jax.numpy as jnp
from jax import lax
from jax.experimental import pallas as pl
from jax.experimental.pallas import tpu as pltpu
from jax.experimental.pallas import tpu_sc as plsc

N_NODES = 10000
N_EDGES = 320000
DIM = 128
EPS = 1e-5

NC = 2
NS = 16
NW = NC * NS

EDGES_PER_W = N_EDGES // NW
CHUNK = 80
N_CHUNKS = EDGES_PER_W // CHUNK
NBUF = 3
N_PAD = 10240
ROWS_PER_TILE = N_PAD // NS
WB_ROWS = CHUNK


def _sc_scatter_add(dst, e):
    mesh = plsc.VectorSubcoreMesh(core_axis_name="c", subcore_axis_name="s")

    @functools.partial(
        pl.kernel,
        out_type=jax.ShapeDtypeStruct((NC, N_PAD, DIM), jnp.float32),
        mesh=mesh,
        scratch_types=[
            pltpu.VMEM((N_CHUNKS, CHUNK), jnp.int32),
            pltpu.VMEM((NBUF, CHUNK, DIM), jnp.float32),
            pltpu.VMEM_SHARED((N_PAD, DIM), jnp.float32),
            pltpu.SemaphoreType.DMA((NBUF,)),
            pltpu.SemaphoreType.DMA((NBUF,)),
            pltpu.SemaphoreType.DMA,
            pltpu.SemaphoreType.DMA,
        ],
    )
    def body(dst_hbm, e_hbm, out_hbm, idx_all, rows, acc,
             sem_g, sem_s, sem_z, sem_i):
        cid = lax.axis_index("c")
        sid = lax.axis_index("s")
        wid = sid * NC + cid
        base = wid * EDGES_PER_W
        rbase = sid * ROWS_PER_TILE

        def fire_gather(ci, b):
            pltpu.async_copy(e_hbm.at[pl.ds(base + ci * CHUNK, CHUNK)],
                             rows.at[b], sem_g.at[b])

        def drain_gather(ci, b):
            pltpu.make_async_copy(e_hbm.at[pl.ds(base + ci * CHUNK, CHUNK)],
                                  rows.at[b], sem_g.at[b]).wait()

        def fire_scatter(ci, b):
            pltpu.async_copy(rows.at[b], acc.at[idx_all.at[ci]], sem_s.at[b],
                             add=True)

        def drain_scatter(ci, b):
            pltpu.make_async_copy(rows.at[b], acc.at[idx_all.at[ci]],
                                  sem_s.at[b]).wait()

        pltpu.async_copy(dst_hbm.at[wid], idx_all, sem_i)
        for c in range(1, NBUF):
            fire_gather(c, c)

        zeros16 = jnp.zeros((16,), jnp.float32)

        def zstore(i, carry):
            r = i // (DIM // 16)
            c = (i % (DIM // 16)) * 16
            rows[0, r, pl.ds(c, 16)] = zeros16
            return carry

        lax.fori_loop(0, CHUNK * (DIM // 16), zstore, 0)

        def zfire(k, carry):
            pltpu.async_copy(rows.at[0],
                             acc.at[pl.ds(rbase + k * CHUNK, CHUNK)], sem_z)
            return carry

        lax.fori_loop(0, ROWS_PER_TILE // CHUNK, zfire, 0)

        def zdrain(k, carry):
            pltpu.make_async_copy(
                rows.at[0], acc.at[pl.ds(rbase + k * CHUNK, CHUNK)],
                sem_z).wait()
            return carry

        lax.fori_loop(0, ROWS_PER_TILE // CHUNK, zdrain, 0)
        fire_gather(0, 0)
        pltpu.make_async_copy(dst_hbm.at[wid], idx_all, sem_i).wait()
        plsc.subcore_barrier()

        def chunk_body(c, carry):
            b = lax.rem(c, NBUF)
            drain_gather(c, b)
            fire_scatter(c, b)

            @pl.when(c >= 1)
            def _():
                b1 = lax.rem(c - 1, NBUF)

                @pl.when(c + NBUF - 1 < N_CHUNKS)
                def _():
                    drain_scatter(c - 1, b1)
                    fire_gather(c + NBUF - 1, b1)

            return carry

        lax.fori_loop(0, N_CHUNKS, chunk_body, 0)
        for c in range(N_CHUNKS - NBUF, N_CHUNKS):
            drain_scatter(c, c % NBUF)
        plsc.subcore_barrier()

        n_wb = ROWS_PER_TILE // WB_ROWS
        for k in range(n_wb):
            b = k % 2
            r = rbase + k * WB_ROWS
            if k >= 2:
                pltpu.make_async_copy(
                    rows.at[b],
                    out_hbm.at[cid, pl.ds(rbase + (k - 2) * WB_ROWS, WB_ROWS)],
                    sem_s.at[b]).wait()
            pltpu.async_copy(acc.at[pl.ds(r, WB_ROWS)], rows.at[b],
                             sem_g.at[b])
            pltpu.make_async_copy(acc.at[pl.ds(r, WB_ROWS)], rows.at[b],
                                  sem_g.at[b]).wait()
            pltpu.async_copy(rows.at[b], out_hbm.at[cid, pl.ds(r, WB_ROWS)],
                             sem_s.at[b])
        for k in range(n_wb - 2, n_wb):
            b = k % 2
            pltpu.make_async_copy(
                rows.at[b],
                out_hbm.at[cid, pl.ds(rbase + k * WB_ROWS, WB_ROWS)],
                sem_s.at[b]).wait()

    return body(dst, e)


BR = 2000


def _mlp_block(x_ref, p_ref, w0x_ref, w0m_ref, w1_ref, w2_ref,
               b0_ref, b1_ref, b2_ref, gnw_ref, gnb_ref, out_ref):
    x = x_ref[...]
    msg = p_ref[0] + p_ref[1]
    h = jnp.dot(x, w0x_ref[...], preferred_element_type=jnp.float32)
    h += jnp.dot(msg, w0m_ref[...], preferred_element_type=jnp.float32)
    h = jnp.maximum(h + b0_ref[...], 0.0)
    h = jnp.dot(h, w1_ref[...], preferred_element_type=jnp.float32)
    h = jnp.maximum(h + b1_ref[...], 0.0)
    h = jnp.dot(h, w2_ref[...], preferred_element_type=jnp.float32)
    h = h + b2_ref[...]
    mean = jnp.mean(h, axis=1, keepdims=True)
    var = jnp.mean((h - mean) ** 2, axis=1, keepdims=True)
    h = (h - mean) * lax.rsqrt(var + EPS) * gnw_ref[...] + gnb_ref[...]
    out_ref[...] = x + h


def _mlp(x, partials, w0x, w0m, w1t, w2t, b0, b1, b2, gn_w, gn_b):
    n = x.shape[0]
    grid = (n // BR,)
    row_spec = pl.BlockSpec((BR, DIM), lambda i: (i, 0))
    p_spec = pl.BlockSpec((NC, BR, DIM), lambda i: (0, i, 0))
    full = lambda a: pl.BlockSpec(a.shape, lambda i: (0,) * a.ndim)
    return pl.pallas_call(
        _mlp_block,
        grid=grid,
        in_specs=[row_spec, p_spec,
                  full(w0x), full(w0m), full(w1t), full(w2t),
                  full(b0), full(b1), full(b2), full(gn_w), full(gn_b)],
        out_specs=row_spec,
        out_shape=jax.ShapeDtypeStruct((n, DIM), jnp.float32),
        compiler_params=pltpu.CompilerParams(
            dimension_semantics=("parallel",),
        ),
    )(x, partials, w0x, w0m, w1t, w2t, b0, b1, b2, gn_w, gn_b)


def kernel(x, edge_index, e, W0, b0, W1, b1, W2, b2, gn_w, gn_b):
    dst = edge_index[1].reshape(NW, N_CHUNKS, CHUNK)
    partials = _sc_scatter_add(dst, e)
    w0t = W0.T
    out = _mlp(x, partials,
               w0t[:DIM], w0t[DIM:], W1.T, W2.T,
               b0[None, :], b1[None, :], b2[None, :],
               gn_w[None, :], gn_b[None, :])
    return out


# speedup vs baseline: 1.0822x; 1.0084x over previous
import functools

import jax
import jax.numpy as jnp
from jax import lax
from jax.experimental import pallas as pl
from jax.experimental.pallas import tpu as pltpu
from jax.experimental.pallas import tpu_sc as plsc

N_NODES = 10000
N_EDGES = 320000
DIM = 128
EPS = 1e-5

NC = 2
NS = 16
NW = NC * NS

EDGES_PER_W = N_EDGES // NW
CHUNK = 80
N_CHUNKS = EDGES_PER_W // CHUNK
NBUF = 3
N_PAD = 10240
ROWS_PER_TILE = N_PAD // NS
WB_ROWS = CHUNK


def _sc_scatter_add(dst, e):
    mesh = plsc.VectorSubcoreMesh(core_axis_name="c", subcore_axis_name="s")

    @functools.partial(
        pl.kernel,
        out_type=jax.ShapeDtypeStruct((NC, N_PAD, DIM), jnp.float32),
        mesh=mesh,
        scratch_types=[
            pltpu.VMEM((N_CHUNKS, CHUNK), jnp.int32),
            pltpu.VMEM((NBUF, CHUNK, DIM), jnp.float32),
            pltpu.VMEM_SHARED((N_PAD, DIM), jnp.float32),
            pltpu.SemaphoreType.DMA((NBUF,)),
            pltpu.SemaphoreType.DMA((NBUF,)),
            pltpu.SemaphoreType.DMA,
            pltpu.SemaphoreType.DMA,
        ],
    )
    def body(dst_hbm, e_hbm, out_hbm, idx_all, rows, acc,
             sem_g, sem_s, sem_z, sem_i):
        cid = lax.axis_index("c")
        sid = lax.axis_index("s")
        wid = sid * NC + cid
        base = wid * EDGES_PER_W
        rbase = sid * ROWS_PER_TILE

        def fire_gather(ci, b):
            pltpu.async_copy(e_hbm.at[pl.ds(base + ci * CHUNK, CHUNK)],
                             rows.at[b], sem_g.at[b])

        def drain_gather(ci, b):
            pltpu.make_async_copy(e_hbm.at[pl.ds(base + ci * CHUNK, CHUNK)],
                                  rows.at[b], sem_g.at[b]).wait()

        def fire_scatter(ci, b):
            pltpu.async_copy(rows.at[b], acc.at[idx_all.at[ci]], sem_s.at[b],
                             add=True)

        def drain_scatter(ci, b):
            pltpu.make_async_copy(rows.at[b], acc.at[idx_all.at[ci]],
                                  sem_s.at[b]).wait()

        pltpu.async_copy(dst_hbm.at[wid], idx_all, sem_i)
        for c in range(1, NBUF):
            fire_gather(c, c)

        zeros16 = jnp.zeros((16,), jnp.float32)

        def zstore(i, carry):
            r = i // (DIM // 16)
            c = (i % (DIM // 16)) * 16
            rows[0, r, pl.ds(c, 16)] = zeros16
            return carry

        lax.fori_loop(0, CHUNK * (DIM // 16), zstore, 0)

        def zfire(k, carry):
            pltpu.async_copy(rows.at[0],
                             acc.at[pl.ds(rbase + k * CHUNK, CHUNK)], sem_z)
            return carry

        lax.fori_loop(0, ROWS_PER_TILE // CHUNK, zfire, 0)

        def zdrain(k, carry):
            pltpu.make_async_copy(
                rows.at[0], acc.at[pl.ds(rbase + k * CHUNK, CHUNK)],
                sem_z).wait()
            return carry

        lax.fori_loop(0, ROWS_PER_TILE // CHUNK, zdrain, 0)
        fire_gather(0, 0)
        pltpu.make_async_copy(dst_hbm.at[wid], idx_all, sem_i).wait()
        plsc.subcore_barrier()

        def chunk_body(c, carry):
            b = lax.rem(c, NBUF)
            drain_gather(c, b)
            fire_scatter(c, b)

            @pl.when(c >= 1)
            def _():
                b1 = lax.rem(c - 1, NBUF)

                @pl.when(c + NBUF - 1 < N_CHUNKS)
                def _():
                    drain_scatter(c - 1, b1)
                    fire_gather(c + NBUF - 1, b1)

            return carry

        lax.fori_loop(0, N_CHUNKS, chunk_body, 0)
        for c in range(N_CHUNKS - NBUF, N_CHUNKS):
            drain_scatter(c, c % NBUF)
        plsc.subcore_barrier()

        n_wb = ROWS_PER_TILE // WB_ROWS
        for k in range(n_wb):
            b = k % 2
            r = rbase + k * WB_ROWS
            if k >= 2:
                pltpu.make_async_copy(
                    rows.at[b],
                    out_hbm.at[cid, pl.ds(rbase + (k - 2) * WB_ROWS, WB_ROWS)],
                    sem_s.at[b]).wait()
            pltpu.async_copy(acc.at[pl.ds(r, WB_ROWS)], rows.at[b],
                             sem_g.at[b])
            pltpu.make_async_copy(acc.at[pl.ds(r, WB_ROWS)], rows.at[b],
                                  sem_g.at[b]).wait()
            pltpu.async_copy(rows.at[b], out_hbm.at[cid, pl.ds(r, WB_ROWS)],
                             sem_s.at[b])
        for k in range(n_wb - 2, n_wb):
            b = k % 2
            pltpu.make_async_copy(
                rows.at[b],
                out_hbm.at[cid, pl.ds(rbase + k * WB_ROWS, WB_ROWS)],
                sem_s.at[b]).wait()

    return body(dst, e)


BR = 5000


def _mlp_block(x_ref, p_ref, w0x_ref, w0m_ref, w1_ref, w2_ref,
               b0_ref, b1_ref, b2_ref, gnw_ref, gnb_ref, out_ref):
    x = x_ref[...]
    msg = p_ref[0] + p_ref[1]
    h = jnp.dot(x, w0x_ref[...], preferred_element_type=jnp.float32)
    h += jnp.dot(msg, w0m_ref[...], preferred_element_type=jnp.float32)
    h = jnp.maximum(h + b0_ref[...], 0.0)
    h = jnp.dot(h, w1_ref[...], preferred_element_type=jnp.float32)
    h = jnp.maximum(h + b1_ref[...], 0.0)
    h = jnp.dot(h, w2_ref[...], preferred_element_type=jnp.float32)
    h = h + b2_ref[...]
    mean = jnp.mean(h, axis=1, keepdims=True)
    var = jnp.mean((h - mean) ** 2, axis=1, keepdims=True)
    h = (h - mean) * lax.rsqrt(var + EPS) * gnw_ref[...] + gnb_ref[...]
    out_ref[...] = x + h


def _mlp(x, partials, w0x, w0m, w1t, w2t, b0, b1, b2, gn_w, gn_b):
    n = x.shape[0]
    grid = (n // BR,)
    row_spec = pl.BlockSpec((BR, DIM), lambda i: (i, 0))
    p_spec = pl.BlockSpec((NC, BR, DIM), lambda i: (0, i, 0))
    full = lambda a: pl.BlockSpec(a.shape, lambda i: (0,) * a.ndim)
    return pl.pallas_call(
        _mlp_block,
        grid=grid,
        in_specs=[row_spec, p_spec,
                  full(w0x), full(w0m), full(w1t), full(w2t),
                  full(b0), full(b1), full(b2), full(gn_w), full(gn_b)],
        out_specs=row_spec,
        out_shape=jax.ShapeDtypeStruct((n, DIM), jnp.float32),
        compiler_params=pltpu.CompilerParams(
            dimension_semantics=("parallel",),
        ),
    )(x, partials, w0x, w0m, w1t, w2t, b0, b1, b2, gn_w, gn_b)


def kernel(x, edge_index, e, W0, b0, W1, b1, W2, b2, gn_w, gn_b):
    dst = edge_index[1].reshape(NW, N_CHUNKS, CHUNK)
    partials = _sc_scatter_add(dst, e)
    w0t = W0.T
    out = _mlp(x, partials,
               w0t[:DIM], w0t[DIM:], W1.T, W2.T,
               b0[None, :], b1[None, :], b2[None, :],
               gn_w[None, :], gn_b[None, :])
    return out
